# baseline (speedup 1.0000x reference)
import functools
import math

import jax
import jax.numpy as jnp
from jax import lax
from jax.experimental import pallas as pl
from jax.experimental.pallas import tpu as pltpu
from jax.experimental.pallas import tpu_sc as plsc

EMBED_DIM = 1024
HALF_DIM = EMBED_DIM // 2
PADDING_IDX = 1

NUM_CORES = 2
NUM_SUBCORES = 16
NUM_WORKERS = NUM_CORES * NUM_SUBCORES

TBLK = 128
CHUNK = 32
LANES = 16


def _table_body(o_ref, *, seq_len):
    i = pl.program_id(0)
    row = (
        lax.broadcasted_iota(jnp.int32, (TBLK, 1), 0) + i * TBLK
    ).astype(jnp.float32)
    valid = row < float(seq_len)
    pos = row + float(PADDING_IDX + 1)
    k = lax.broadcasted_iota(jnp.int32, (1, HALF_DIM), 1).astype(jnp.float32)
    inv_freq = jnp.exp(k * (-math.log(10000.0) / (HALF_DIM - 1)))
    ang = pos * inv_freq
    emb = jnp.concatenate([jnp.sin(ang), jnp.cos(ang)], axis=1)
    o_ref[...] = jnp.where(valid, emb, 0.0)


def _build_table(seq_len):
    rows = seq_len + TBLK
    return pl.pallas_call(
        functools.partial(_table_body, seq_len=seq_len),
        out_shape=jax.ShapeDtypeStruct((rows, EMBED_DIM), jnp.float32),
        grid=(rows // TBLK,),
        out_specs=pl.BlockSpec((TBLK, EMBED_DIM), lambda i: (i, 0)),
    )()


def _gather_body(tok_hbm, table_hbm, out_hbm, tok_v, idx_v, buf_v, sems, *,
                 rows_per_worker, seq_len):
    wid = lax.axis_index("s") * NUM_CORES + lax.axis_index("c")
    base = wid * rows_per_worker
    jbase = base % seq_len

    pltpu.sync_copy(tok_hbm.at[pl.ds(base, rows_per_worker)], tok_v)

    zero_row = seq_len
    for i in range(rows_per_worker // LANES):
        tok = tok_v[pl.ds(i * LANES, LANES)]
        jv = lax.broadcasted_iota(jnp.int32, (LANES,), 0) + (jbase + i * LANES)
        idx = jnp.where(tok == PADDING_IDX, zero_row, jv)
        idx_v[pl.ds(i * LANES, LANES)] = idx

    nchunks = rows_per_worker // CHUNK
    copies = [None, None]
    copies[0] = pltpu.make_async_copy(
        table_hbm.at[idx_v.at[pl.ds(0, CHUNK)]], buf_v.at[0], sems.at[0]
    )
    copies[0].start()
    for c in range(nchunks):
        slot = c % 2
        if c + 1 < nchunks:
            nslot = (c + 1) % 2
            copies[nslot] = pltpu.make_async_copy(
                table_hbm.at[idx_v.at[pl.ds((c + 1) * CHUNK, CHUNK)]],
                buf_v.at[nslot],
                sems.at[nslot],
            )
            copies[nslot].start()
        copies[slot].wait()
        pltpu.sync_copy(buf_v.at[slot], out_hbm.at[pl.ds(base + c * CHUNK, CHUNK)])


def _gather(tok_flat, table, rows_per_worker, seq_len):
    total = tok_flat.shape[0]
    mesh = plsc.VectorSubcoreMesh(
        core_axis_name="c",
        subcore_axis_name="s",
        num_cores=NUM_CORES,
        num_subcores=NUM_SUBCORES,
    )
    body = functools.partial(
        _gather_body, rows_per_worker=rows_per_worker, seq_len=seq_len
    )
    return pl.kernel(
        body,
        out_type=jax.ShapeDtypeStruct((total, EMBED_DIM), jnp.float32),
        mesh=mesh,
        scratch_types=[
            pltpu.VMEM((rows_per_worker,), jnp.int32),
            pltpu.VMEM((rows_per_worker,), jnp.int32),
            pltpu.VMEM((2, CHUNK, EMBED_DIM), jnp.float32),
            pltpu.SemaphoreType.DMA((2,)),
        ],
    )(tok_flat, table)


def kernel(tokens):
    bsz, seq_len = tokens.shape
    total = bsz * seq_len
    rows_per_worker = total // NUM_WORKERS
    table = _build_table(seq_len)
    out = _gather(tokens.reshape(-1), table, rows_per_worker, seq_len)
    return out.reshape(bsz, seq_len, EMBED_DIM)
